# baseline (speedup 1.0000x reference)
import functools

import jax
import jax.numpy as jnp
import numpy as np
from jax import lax
from jax.experimental import pallas as pl
from jax.experimental.pallas import tpu as pltpu
from jax.experimental.pallas import tpu_sc as plsc

_FIELD_DIMS = [1000] * 26
_F = len(_FIELD_DIMS)
_V = sum(_FIELD_DIMS)
_D = 32
_B = 4096
_OFFS = np.array((0, *np.cumsum(_FIELD_DIMS)[:-1]), dtype=np.int32)
_FP = 32
_NPAD = _FP - _F
_NTILE = 32
_BPT = _B // _NTILE
_CH = 32
_NCH = _BPT // _CH
_NSTR = 4
_SLEN = _FP // _NSTR


def _ffm_body(table, idxp, wlin, blin, out,
              ix_v, rows0, rows1, wlin_v, blin_v, out_v, sem0, sem1):
    nc = 2
    wid = lax.axis_index("s") * nc + lax.axis_index("c")
    base = wid * _BPT

    pltpu.sync_copy(wlin, wlin_v)
    pltpu.sync_copy(blin, blin_v)
    b0vec = blin_v[pl.ds(0, 16)]
    w0vec = wlin_v[pl.ds(0, 16)]
    lane = lax.iota(jnp.int32, 16)

    def fire(lb, rows_v, sem):
        for c in range(_NSTR):
            pltpu.async_copy(table.at[ix_v.at[lb, pl.ds(c * _SLEN, _SLEN)]],
                             rows_v.at[pl.ds(c * _SLEN, _SLEN)], sem)

    def drain(rows_v, sem):
        for c in range(_NSTR):
            pltpu.make_async_copy(table.at[ix_v.at[0, pl.ds(c * _SLEN, _SLEN)]],
                                  rows_v.at[pl.ds(c * _SLEN, _SLEN)], sem).wait()

    def compute(lb, rows_v, ch, res):
        def ibody(i, acc):
            def jbody(j, acc2):
                lo, hi = plsc.unpack(rows_v[i, pl.ds(j * _D, _D)],
                                     format=plsc.PackFormat.INTERLEAVED)
                ro, rh = plsc.unpack(rows_v[j, pl.ds(i * _D, _D)],
                                     format=plsc.PackFormat.INTERLEAVED)
                return acc2 + lo * ro + hi * rh

            return lax.fori_loop(i + 1, _F, jbody, acc)

        acc = lax.fori_loop(0, _F - 1, ibody, jnp.zeros((16,), jnp.float32))
        g1 = plsc.load_gather(wlin_v, [ix_v[lb, pl.ds(0, 16)]])
        g2 = plsc.load_gather(wlin_v, [ix_v[lb, pl.ds(16, 16)]])
        s = (jnp.sum(acc) + jnp.sum(g1 + g2)
             - jnp.float32(_NPAD) * w0vec[0] + b0vec[0])
        res = jnp.where(lane == (lb & 15), s, res)

        @pl.when((lb & 15) == 15)
        def _():
            out_v[pl.ds(ch * _CH + lb - 15, 16)] = res

        return res

    for ch in range(_NCH):
        b0 = base + ch * _CH
        pltpu.sync_copy(idxp.at[pl.ds(b0, _CH)], ix_v)
        fire(0, rows0, sem0)

        def body2(t, res):
            lb0 = 2 * t
            lb1 = lb0 + 1
            fire(lb1, rows1, sem1)
            drain(rows0, sem0)
            res = compute(lb0, rows0, ch, res)

            @pl.when(lb1 < _CH - 1)
            def _():
                fire(lb0 + 2, rows0, sem0)

            drain(rows1, sem1)
            res = compute(lb1, rows1, ch, res)
            return res

        lax.fori_loop(0, _CH // 2, body2, jnp.zeros((16,), jnp.float32))

    pltpu.sync_copy(out_v, out.at[pl.ds(base, _BPT)])


@jax.jit
def kernel(x, W_lin, b_lin, W_emb):
    offs = jnp.asarray(_OFFS)
    idx = x + offs[None, :]
    idxp = jnp.concatenate(
        [idx, jnp.zeros((_B, _FP - _F), jnp.int32)], axis=1)
    table = jnp.transpose(W_emb, (1, 0, 2)).astype(
        jnp.bfloat16).reshape(_V, _F * _D)
    wlin_pad = jnp.concatenate([W_lin[:, 0], jnp.zeros((8,), jnp.float32)])
    blin_pad = jnp.concatenate([b_lin, jnp.zeros((15,), jnp.float32)])

    mesh = plsc.VectorSubcoreMesh(core_axis_name="c", subcore_axis_name="s")
    run = functools.partial(
        pl.kernel, _ffm_body,
        out_type=jax.ShapeDtypeStruct((_B,), jnp.float32),
        mesh=mesh,
        compiler_params=pltpu.CompilerParams(
            needs_layout_passes=False, use_tc_tiling_on_sc=False),
        scratch_types=[
            pltpu.VMEM((_CH, _FP), jnp.int32),
            pltpu.VMEM((_FP, _F * _D), jnp.bfloat16),
            pltpu.VMEM((_FP, _F * _D), jnp.bfloat16),
            pltpu.VMEM((_V + 8,), jnp.float32),
            pltpu.VMEM((16,), jnp.float32),
            pltpu.VMEM((_BPT,), jnp.float32),
            pltpu.SemaphoreType.DMA,
            pltpu.SemaphoreType.DMA,
        ],
    )()
    return run(table, idxp, wlin_pad, blin_pad)
